# Initial kernel scaffold; baseline (speedup 1.0000x reference)
import jax
import jax.numpy as jnp
from jax.experimental import pallas as pl


def kernel(age, education, emb, W1, b1, W2, b2, W3, b3):
    raise NotImplementedError("write your pallas kernel here")


# speedup vs baseline: 1.5502x; 1.5502x over previous
import functools

import jax
import jax.numpy as jnp
from jax import lax
from jax.experimental import pallas as pl
from jax.experimental.pallas import tpu as pltpu
from jax.experimental.pallas import tpu_sc as plsc

B = 16384
VOCAB = 1000
HID = 10
PAD_W = 16
NC = 2
NS = 16
NW = NC * NS
L = 16
BPW = B // NW
GROUPS = BPW // L
NPAR = 136


def _fold_table_body(emb_ref, w1p_ref, b1p_ref, out_ref):
    out_ref[...] = (
        jnp.dot(emb_ref[...], w1p_ref[...],
                preferred_element_type=jnp.float32,
                precision=lax.Precision.HIGHEST)
        + b1p_ref[...]
    )


def _fold_table(emb, w1p, b1p):
    return pl.pallas_call(
        _fold_table_body,
        out_shape=jax.ShapeDtypeStruct((VOCAB, PAD_W), jnp.float32),
    )(emb, w1p, b1p)


def _sc_body(tb_hbm, par_hbm, age_hbm, edu_hbm, out_hbm,
             tb_v, par_v, age_v, edu_v, out_v):
    cid = lax.axis_index("c")
    sid = lax.axis_index("s")
    wid = sid * NC + cid
    base = wid * BPW

    pltpu.sync_copy(tb_hbm, tb_v)
    pltpu.sync_copy(par_hbm, par_v)
    pltpu.sync_copy(age_hbm.at[pl.ds(base, BPW)], age_v)
    pltpu.sync_copy(edu_hbm.at[pl.ds(base, BPW)], edu_v)

    def group(g, _):
        off = g * L
        edu_g = edu_v[pl.ds(off, L)]
        age_g = age_v[pl.ds(off, L)]
        flat = edu_g * PAD_W
        h1 = []
        for j in range(HID):
            gj = plsc.load_gather(tb_v, [flat + j])
            h1.append(jnp.maximum(age_g * par_v[j] + gj, 0.0))
        h2 = []
        for j in range(HID):
            acc = par_v[110 + j]
            for k in range(HID):
                acc = acc + h1[k] * par_v[10 + k * HID + j]
            h2.append(jnp.maximum(acc, 0.0))
        o = par_v[130]
        for k in range(HID):
            o = o + h2[k] * par_v[120 + k]
        out_v[pl.ds(off, L)] = 1.0 / (1.0 + jnp.exp(-o))
        return 0

    lax.fori_loop(0, GROUPS, group, 0)
    pltpu.sync_copy(out_v, out_hbm.at[pl.ds(base, BPW)])


@functools.cache
def _sc_mlp():
    return functools.partial(
        pl.kernel,
        out_type=jax.ShapeDtypeStruct((B,), jnp.float32),
        mesh=plsc.VectorSubcoreMesh(core_axis_name="c", subcore_axis_name="s",
                                    num_cores=NC, num_subcores=NS),
        scratch_types=[
            pltpu.VMEM((VOCAB * PAD_W,), jnp.float32),
            pltpu.VMEM((NPAR, L), jnp.float32),
            pltpu.VMEM((BPW,), jnp.float32),
            pltpu.VMEM((BPW,), jnp.int32),
            pltpu.VMEM((BPW,), jnp.float32),
        ],
        compiler_params=pltpu.CompilerParams(needs_layout_passes=False),
    )(_sc_body)


def kernel(age, education, emb, W1, b1, W2, b2, W3, b3):
    w1p = jnp.concatenate([W1[1:], jnp.zeros((emb.shape[1], PAD_W - HID),
                                             jnp.float32)], axis=1)
    b1p = jnp.concatenate([b1, jnp.zeros((PAD_W - HID,), jnp.float32)])[None, :]
    tb = _fold_table(emb, w1p, b1p)

    pars = jnp.concatenate([
        W1[0, :], W2.reshape(-1), b2, W3[:, 0], b3,
        jnp.zeros((NPAR - 131,), jnp.float32),
    ])
    par2d = jnp.broadcast_to(pars[:, None], (NPAR, L))

    age_f = age.astype(jnp.float32).reshape(B)
    edu = education.reshape(B).astype(jnp.int32)

    out = _sc_mlp()(tb.reshape(VOCAB * PAD_W), par2d, age_f, edu)
    return out.reshape(B, 1)
